# baseline (speedup 1.0000x reference)
import dataclasses
import functools

import jax
import jax.numpy as jnp
from jax import lax
from jax.experimental import pallas as pl
from jax.experimental.pallas import tpu as pltpu
from jax.experimental.pallas import tpu_sc as plsc

N = 10000
E = 320000
D = 128
OUT = 128

NC = 2
NS = 16
NW = NC * NS
CH = 128
CPW = 80
SB = 8
NCHUNKS = NW * CPW
E_PAD = NCHUNKS * CH
STEP = 2 * SB
ROWS_PER_SUB = 624
TAIL_ROWS = N - NS * ROWS_PER_SUB
LANES = 16
EPI = 8


def _sc_aggregate(x, src2, dst2, wt2, zeros):
    mesh = plsc.VectorSubcoreMesh(core_axis_name="c", subcore_axis_name="s")
    cp = pltpu.CompilerParams()
    if "needs_layout_passes" in pltpu.CompilerParams.__dataclass_fields__:
        cp = dataclasses.replace(cp, needs_layout_passes=False)

    @functools.partial(
        pl.kernel,
        out_type=jax.ShapeDtypeStruct((NC, N, D), jnp.float32),
        mesh=mesh,
        compiler_params=cp,
        scratch_types=[
            pltpu.VMEM((SB, CH), jnp.int32),
            pltpu.VMEM((SB, CH), jnp.int32),
            pltpu.VMEM((SB, CH), jnp.float32),
            pltpu.VMEM((CH, D), jnp.float32),
            pltpu.VMEM_SHARED((N, D), jnp.float32),
            pltpu.SemaphoreType.DMA,
        ],
    )
    def agg(x_hbm, src_hbm, dst_hbm, wt_hbm, z_hbm, hp_hbm,
            srcb, dstb, wtb, rowb, h_sh, gsem):
        cid = lax.axis_index("c")
        sid = lax.axis_index("s")
        wid = sid * NC + cid
        row0 = sid * ROWS_PER_SUB
        base = wid * CPW

        pltpu.sync_copy(z_hbm.at[pl.ds(row0, ROWS_PER_SUB)],
                        h_sh.at[pl.ds(row0, ROWS_PER_SUB)])

        @pl.when(sid == NS - 1)
        def _zero_tail():
            pltpu.sync_copy(z_hbm.at[pl.ds(NS * ROWS_PER_SUB, TAIL_ROWS)],
                            h_sh.at[pl.ds(NS * ROWS_PER_SUB, TAIL_ROWS)])

        plsc.subcore_barrier()

        def scale_rows(j):
            jv = jnp.full((LANES,), j, jnp.int32)

            @pl.loop(0, CH // EPI)
            def _it(it):
                for jj in range(EPI):
                    e = it * EPI + jj
                    w = plsc.load_gather(
                        wtb, [jv, jnp.full((LANES,), e, jnp.int32)])
                    for dd in range(D // LANES):
                        sl = pl.ds(dd * LANES, LANES)
                        rowb[e, sl] = rowb[e, sl] * w

        @pl.loop(0, CPW // SB)
        def _super(sb):
            sl = pl.ds(base + sb * SB, SB)
            pltpu.sync_copy(src_hbm.at[sl], srcb)
            pltpu.sync_copy(dst_hbm.at[sl], dstb)
            pltpu.sync_copy(wt_hbm.at[sl], wtb)
            for j in range(SB):
                pltpu.async_copy(x_hbm.at[srcb.at[j]], rowb, gsem).wait()
                scale_rows(j)
                pltpu.sync_copy(rowb, h_sh.at[dstb.at[j]], add=True)

        plsc.subcore_barrier()
        pltpu.sync_copy(h_sh.at[pl.ds(row0, ROWS_PER_SUB)],
                        hp_hbm.at[cid, pl.ds(row0, ROWS_PER_SUB)])

        @pl.when(sid == NS - 1)
        def _flush_tail():
            pltpu.sync_copy(h_sh.at[pl.ds(NS * ROWS_PER_SUB, TAIL_ROWS)],
                            hp_hbm.at[cid, pl.ds(NS * ROWS_PER_SUB, TAIL_ROWS)])

    return agg(x, src2, dst2, wt2, zeros)


def _tc_project(h0, h1, x, wt, wb):
    RB = 1000

    def body(h0_ref, h1_ref, x_ref, wt_ref, wb_ref, o_ref):
        h = h0_ref[...] + h1_ref[...]
        acc = jnp.dot(h, wt_ref[...], preferred_element_type=jnp.float32)
        acc = acc + jnp.dot(x_ref[...], wb_ref[...],
                            preferred_element_type=jnp.float32)
        o_ref[...] = jnp.maximum(acc, 0.0)

    return pl.pallas_call(
        body,
        grid=(N // RB,),
        in_specs=[
            pl.BlockSpec((RB, D), lambda i: (i, 0)),
            pl.BlockSpec((RB, D), lambda i: (i, 0)),
            pl.BlockSpec((RB, D), lambda i: (i, 0)),
            pl.BlockSpec((D, OUT), lambda i: (0, 0)),
            pl.BlockSpec((D, OUT), lambda i: (0, 0)),
        ],
        out_specs=pl.BlockSpec((RB, OUT), lambda i: (i, 0)),
        out_shape=jax.ShapeDtypeStruct((N, OUT), jnp.float32),
    )(h0, h1, x, wt, wb)


def kernel(x, edge_index, edge_weight, W):
    pad = E_PAD - E
    src2 = jnp.concatenate(
        [edge_index[1], jnp.zeros((pad,), jnp.int32)]).reshape(NCHUNKS, CH)
    dst2 = jnp.concatenate(
        [edge_index[0], jnp.zeros((pad,), jnp.int32)]).reshape(NCHUNKS, CH)
    wt2 = jnp.concatenate(
        [edge_weight, jnp.zeros((pad,), jnp.float32)]).reshape(NCHUNKS, CH)
    zeros = jnp.zeros((N, D), jnp.float32)
    hp = _sc_aggregate(x, src2, dst2, wt2, zeros)
    return _tc_project(hp[0], hp[1], x, W[:D], W[D:])


# speedup vs baseline: 1.7431x; 1.7431x over previous
import dataclasses
import functools

import jax
import jax.numpy as jnp
from jax import lax
from jax.experimental import pallas as pl
from jax.experimental.pallas import tpu as pltpu
from jax.experimental.pallas import tpu_sc as plsc

N = 10000
E = 320000
D = 128
OUT = 128

NC = 2
NS = 16
NW = NC * NS
CH = 128
NCHUNKS = E // CH
ROWS_PER_SUB = 624
TAIL_ROWS = N - NS * ROWS_PER_SUB
LANES = 16


def _sc_aggregate(x, src2, dst2, wt2, zeros):
    mesh = plsc.VectorSubcoreMesh(core_axis_name="c", subcore_axis_name="s")
    cp = pltpu.CompilerParams()
    if "needs_layout_passes" in pltpu.CompilerParams.__dataclass_fields__:
        cp = dataclasses.replace(cp, needs_layout_passes=False)

    @functools.partial(
        pl.kernel,
        out_type=jax.ShapeDtypeStruct((NC, N, D), jnp.float32),
        mesh=mesh,
        compiler_params=cp,
        scratch_types=[
            pltpu.VMEM((CH,), jnp.int32),
            pltpu.VMEM((CH,), jnp.int32),
            pltpu.VMEM((CH,), jnp.float32),
            pltpu.VMEM((CH, D), jnp.float32),
            pltpu.VMEM_SHARED((N, D), jnp.float32),
            pltpu.SemaphoreType.DMA,
        ],
    )
    def agg(x_hbm, src_hbm, dst_hbm, wt_hbm, z_hbm, hp_hbm,
            src_v, dst_v, wt_v, rows_v, h_sh, sem):
        cid = lax.axis_index("c")
        sid = lax.axis_index("s")
        wid = sid * NC + cid
        row0 = sid * ROWS_PER_SUB

        pltpu.sync_copy(z_hbm.at[pl.ds(row0, ROWS_PER_SUB)],
                        h_sh.at[pl.ds(row0, ROWS_PER_SUB)])

        @pl.when(sid == NS - 1)
        def _zero_tail():
            pltpu.sync_copy(z_hbm.at[pl.ds(NS * ROWS_PER_SUB, TAIL_ROWS)],
                            h_sh.at[pl.ds(NS * ROWS_PER_SUB, TAIL_ROWS)])

        plsc.subcore_barrier()

        @pl.loop(wid, NCHUNKS, step=NW)
        def _chunk(c):
            pltpu.sync_copy(src_hbm.at[c], src_v)
            pltpu.sync_copy(dst_hbm.at[c], dst_v)
            pltpu.sync_copy(wt_hbm.at[c], wt_v)
            pltpu.async_copy(x_hbm.at[src_v], rows_v, sem).wait()

            @pl.loop(0, CH)
            def _edge(e):
                w = plsc.load_gather(wt_v, [jnp.full((LANES,), e, jnp.int32)])
                for dd in range(D // LANES):
                    sl = pl.ds(dd * LANES, LANES)
                    rows_v[e, sl] = rows_v[e, sl] * w

            pltpu.sync_copy(rows_v, h_sh.at[dst_v], add=True)

        plsc.subcore_barrier()
        pltpu.sync_copy(h_sh.at[pl.ds(row0, ROWS_PER_SUB)],
                        hp_hbm.at[cid, pl.ds(row0, ROWS_PER_SUB)])

        @pl.when(sid == NS - 1)
        def _flush_tail():
            pltpu.sync_copy(h_sh.at[pl.ds(NS * ROWS_PER_SUB, TAIL_ROWS)],
                            hp_hbm.at[cid, pl.ds(NS * ROWS_PER_SUB, TAIL_ROWS)])

    return agg(x, src2, dst2, wt2, zeros)


def _tc_project(h0, h1, x, wt, wb):
    RB = 1000

    def body(h0_ref, h1_ref, x_ref, wt_ref, wb_ref, o_ref):
        h = h0_ref[...] + h1_ref[...]
        acc = jnp.dot(h, wt_ref[...], preferred_element_type=jnp.float32)
        acc = acc + jnp.dot(x_ref[...], wb_ref[...],
                            preferred_element_type=jnp.float32)
        o_ref[...] = jnp.maximum(acc, 0.0)

    return pl.pallas_call(
        body,
        grid=(N // RB,),
        in_specs=[
            pl.BlockSpec((RB, D), lambda i: (i, 0)),
            pl.BlockSpec((RB, D), lambda i: (i, 0)),
            pl.BlockSpec((RB, D), lambda i: (i, 0)),
            pl.BlockSpec((D, OUT), lambda i: (0, 0)),
            pl.BlockSpec((D, OUT), lambda i: (0, 0)),
        ],
        out_specs=pl.BlockSpec((RB, OUT), lambda i: (i, 0)),
        out_shape=jax.ShapeDtypeStruct((N, OUT), jnp.float32),
    )(h0, h1, x, wt, wb)


def kernel(x, edge_index, edge_weight, W):
    src2 = edge_index[1].reshape(NCHUNKS, CH)
    dst2 = edge_index[0].reshape(NCHUNKS, CH)
    wt2 = edge_weight.reshape(NCHUNKS, CH)
    zeros = jnp.zeros((N, D), jnp.float32)
    hp = _sc_aggregate(x, src2, dst2, wt2, zeros)
    return _tc_project(hp[0], hp[1], x, W[:D], W[D:])


# speedup vs baseline: 1.7807x; 1.0216x over previous
import dataclasses
import functools

import jax
import jax.numpy as jnp
from jax import lax
from jax.experimental import pallas as pl
from jax.experimental.pallas import tpu as pltpu
from jax.experimental.pallas import tpu_sc as plsc

N = 10000
E = 320000
D = 128
OUT = 128

NC = 2
NS = 16
NW = NC * NS
CH = 128
NCHUNKS = E // CH
ROWS_PER_SUB = 624
TAIL_ROWS = N - NS * ROWS_PER_SUB
LANES = 16


def _sc_aggregate(x, src2, dst2, wt2, zeros):
    mesh = plsc.VectorSubcoreMesh(core_axis_name="c", subcore_axis_name="s")
    cp = pltpu.CompilerParams()
    if "needs_layout_passes" in pltpu.CompilerParams.__dataclass_fields__:
        cp = dataclasses.replace(cp, needs_layout_passes=False)

    @functools.partial(
        pl.kernel,
        out_type=jax.ShapeDtypeStruct((NC, N, D), jnp.float32),
        mesh=mesh,
        compiler_params=cp,
        scratch_types=[
            pltpu.VMEM((CH,), jnp.int32),
            pltpu.VMEM((CH,), jnp.int32),
            pltpu.VMEM((CH,), jnp.float32),
            pltpu.VMEM((CH, D), jnp.float32),
            pltpu.VMEM_SHARED((N, D), jnp.float32),
            pltpu.SemaphoreType.DMA,
        ],
    )
    def agg(x_hbm, src_hbm, dst_hbm, wt_hbm, z_hbm, hp_hbm,
            src_v, dst_v, wt_v, rows_v, h_sh, sem):
        cid = lax.axis_index("c")
        sid = lax.axis_index("s")
        wid = sid * NC + cid
        row0 = sid * ROWS_PER_SUB

        pltpu.sync_copy(z_hbm.at[pl.ds(row0, ROWS_PER_SUB)],
                        h_sh.at[pl.ds(row0, ROWS_PER_SUB)])

        @pl.when(sid == NS - 1)
        def _zero_tail():
            pltpu.sync_copy(z_hbm.at[pl.ds(NS * ROWS_PER_SUB, TAIL_ROWS)],
                            h_sh.at[pl.ds(NS * ROWS_PER_SUB, TAIL_ROWS)])

        plsc.subcore_barrier()

        @pl.loop(wid, NCHUNKS, step=NW)
        def _chunk(c):
            pltpu.sync_copy(src_hbm.at[c], src_v)
            pltpu.sync_copy(dst_hbm.at[c], dst_v)
            pltpu.sync_copy(wt_hbm.at[c], wt_v)
            pltpu.async_copy(x_hbm.at[src_v], rows_v, sem).wait()

            @pl.loop(0, CH, step=4)
            def _edge(e0):
                for jj in range(4):
                    e = e0 + jj
                    w = plsc.load_gather(
                        wt_v, [jnp.full((LANES,), e, jnp.int32)])
                    for dd in range(D // LANES):
                        sl = pl.ds(dd * LANES, LANES)
                        rows_v[e, sl] = rows_v[e, sl] * w

            pltpu.sync_copy(rows_v, h_sh.at[dst_v], add=True)

        plsc.subcore_barrier()
        pltpu.sync_copy(h_sh.at[pl.ds(row0, ROWS_PER_SUB)],
                        hp_hbm.at[cid, pl.ds(row0, ROWS_PER_SUB)])

        @pl.when(sid == NS - 1)
        def _flush_tail():
            pltpu.sync_copy(h_sh.at[pl.ds(NS * ROWS_PER_SUB, TAIL_ROWS)],
                            hp_hbm.at[cid, pl.ds(NS * ROWS_PER_SUB, TAIL_ROWS)])

    return agg(x, src2, dst2, wt2, zeros)


def _tc_project(h0, h1, x, wt, wb):
    RB = 1000

    def body(h0_ref, h1_ref, x_ref, wt_ref, wb_ref, o_ref):
        h = h0_ref[...] + h1_ref[...]
        acc = jnp.dot(h, wt_ref[...], preferred_element_type=jnp.float32)
        acc = acc + jnp.dot(x_ref[...], wb_ref[...],
                            preferred_element_type=jnp.float32)
        o_ref[...] = jnp.maximum(acc, 0.0)

    return pl.pallas_call(
        body,
        grid=(N // RB,),
        in_specs=[
            pl.BlockSpec((RB, D), lambda i: (i, 0)),
            pl.BlockSpec((RB, D), lambda i: (i, 0)),
            pl.BlockSpec((RB, D), lambda i: (i, 0)),
            pl.BlockSpec((D, OUT), lambda i: (0, 0)),
            pl.BlockSpec((D, OUT), lambda i: (0, 0)),
        ],
        out_specs=pl.BlockSpec((RB, OUT), lambda i: (i, 0)),
        out_shape=jax.ShapeDtypeStruct((N, OUT), jnp.float32),
    )(h0, h1, x, wt, wb)


def kernel(x, edge_index, edge_weight, W):
    src2 = edge_index[1].reshape(NCHUNKS, CH)
    dst2 = edge_index[0].reshape(NCHUNKS, CH)
    wt2 = edge_weight.reshape(NCHUNKS, CH)
    zeros = jnp.zeros((N, D), jnp.float32)
    hp = _sc_aggregate(x, src2, dst2, wt2, zeros)
    return _tc_project(hp[0], hp[1], x, W[:D], W[D:])


# speedup vs baseline: 2.0137x; 1.1308x over previous
import dataclasses
import functools

import jax
import jax.numpy as jnp
from jax import lax
from jax.experimental import pallas as pl
from jax.experimental.pallas import tpu as pltpu
from jax.experimental.pallas import tpu_sc as plsc

N = 10000
E = 320000
D = 128
OUT = 128

NC = 2
NS = 16
NW = NC * NS
CH = 128
NCHUNKS = E // CH
ROWS_PER_SUB = 624
TAIL_ROWS = N - NS * ROWS_PER_SUB
LANES = 16


def _sc_aggregate(x, src2, dst2, wt2, zeros):
    mesh = plsc.VectorSubcoreMesh(core_axis_name="c", subcore_axis_name="s")
    cp = pltpu.CompilerParams()
    if "needs_layout_passes" in pltpu.CompilerParams.__dataclass_fields__:
        cp = dataclasses.replace(cp, needs_layout_passes=False)

    @functools.partial(
        pl.kernel,
        out_type=jax.ShapeDtypeStruct((NC, N, D), jnp.float32),
        mesh=mesh,
        compiler_params=cp,
        scratch_types=[
            pltpu.VMEM((CH,), jnp.int32),
            pltpu.VMEM((CH,), jnp.int32),
            pltpu.VMEM((CH,), jnp.float32),
            pltpu.VMEM((CH,), jnp.int32),
            pltpu.VMEM((CH,), jnp.int32),
            pltpu.VMEM((CH,), jnp.float32),
            pltpu.VMEM((CH, D), jnp.float32),
            pltpu.VMEM((CH, D), jnp.float32),
            pltpu.VMEM_SHARED((N, D), jnp.float32),
            pltpu.SemaphoreType.DMA,
            pltpu.SemaphoreType.DMA,
            pltpu.SemaphoreType.DMA,
        ],
    )
    def agg(x_hbm, src_hbm, dst_hbm, wt_hbm, z_hbm, hp_hbm,
            src_a, dst_a, wt_a, src_b, dst_b, wt_b, rows_a, rows_b,
            h_sh, sem, ssem_a, ssem_b):
        cid = lax.axis_index("c")
        sid = lax.axis_index("s")
        wid = sid * NC + cid
        row0 = sid * ROWS_PER_SUB

        pltpu.sync_copy(z_hbm.at[pl.ds(row0, ROWS_PER_SUB)],
                        h_sh.at[pl.ds(row0, ROWS_PER_SUB)])

        @pl.when(sid == NS - 1)
        def _zero_tail():
            pltpu.sync_copy(z_hbm.at[pl.ds(NS * ROWS_PER_SUB, TAIL_ROWS)],
                            h_sh.at[pl.ds(NS * ROWS_PER_SUB, TAIL_ROWS)])

        plsc.subcore_barrier()

        def scale_rows(rows_v, wt_v):
            @pl.loop(0, CH, step=4)
            def _edge(e0):
                for jj in range(4):
                    e = e0 + jj
                    w = plsc.load_gather(
                        wt_v, [jnp.full((LANES,), e, jnp.int32)])
                    for dd in range(D // LANES):
                        sl = pl.ds(dd * LANES, LANES)
                        rows_v[e, sl] = rows_v[e, sl] * w

        @pl.loop(wid, NCHUNKS - NW, step=2 * NW)
        def _pair(c):
            pltpu.sync_copy(src_hbm.at[c], src_a)
            pltpu.sync_copy(dst_hbm.at[c], dst_a)
            pltpu.sync_copy(wt_hbm.at[c], wt_a)
            pltpu.async_copy(x_hbm.at[src_a], rows_a, sem).wait()

            @pl.when(c != wid)
            def _drain_prev():
                pltpu.make_async_copy(rows_b, h_sh.at[dst_b], ssem_b).wait()

            scale_rows(rows_a, wt_a)
            pltpu.async_copy(rows_a, h_sh.at[dst_a], ssem_a, add=True)

            cb = c + NW
            pltpu.sync_copy(src_hbm.at[cb], src_b)
            pltpu.sync_copy(dst_hbm.at[cb], dst_b)
            pltpu.sync_copy(wt_hbm.at[cb], wt_b)
            pltpu.async_copy(x_hbm.at[src_b], rows_b, sem).wait()
            pltpu.make_async_copy(rows_a, h_sh.at[dst_a], ssem_a).wait()
            scale_rows(rows_b, wt_b)
            pltpu.async_copy(rows_b, h_sh.at[dst_b], ssem_b, add=True)

        pltpu.make_async_copy(rows_b, h_sh.at[dst_b], ssem_b).wait()

        @pl.when(wid < (NCHUNKS - NW * (NCHUNKS // NW)))
        def _tail_chunk():
            c = NW * (NCHUNKS // NW) + wid
            pltpu.sync_copy(src_hbm.at[c], src_a)
            pltpu.sync_copy(dst_hbm.at[c], dst_a)
            pltpu.sync_copy(wt_hbm.at[c], wt_a)
            pltpu.async_copy(x_hbm.at[src_a], rows_a, sem).wait()
            scale_rows(rows_a, wt_a)
            pltpu.sync_copy(rows_a, h_sh.at[dst_a], add=True)

        plsc.subcore_barrier()
        pltpu.sync_copy(h_sh.at[pl.ds(row0, ROWS_PER_SUB)],
                        hp_hbm.at[cid, pl.ds(row0, ROWS_PER_SUB)])

        @pl.when(sid == NS - 1)
        def _flush_tail():
            pltpu.sync_copy(h_sh.at[pl.ds(NS * ROWS_PER_SUB, TAIL_ROWS)],
                            hp_hbm.at[cid, pl.ds(NS * ROWS_PER_SUB, TAIL_ROWS)])

    return agg(x, src2, dst2, wt2, zeros)


def _tc_project(h0, h1, x, wt, wb):
    RB = 1000

    def body(h0_ref, h1_ref, x_ref, wt_ref, wb_ref, o_ref):
        h = h0_ref[...] + h1_ref[...]
        acc = jnp.dot(h, wt_ref[...], preferred_element_type=jnp.float32)
        acc = acc + jnp.dot(x_ref[...], wb_ref[...],
                            preferred_element_type=jnp.float32)
        o_ref[...] = jnp.maximum(acc, 0.0)

    return pl.pallas_call(
        body,
        grid=(N // RB,),
        in_specs=[
            pl.BlockSpec((RB, D), lambda i: (i, 0)),
            pl.BlockSpec((RB, D), lambda i: (i, 0)),
            pl.BlockSpec((RB, D), lambda i: (i, 0)),
            pl.BlockSpec((D, OUT), lambda i: (0, 0)),
            pl.BlockSpec((D, OUT), lambda i: (0, 0)),
        ],
        out_specs=pl.BlockSpec((RB, OUT), lambda i: (i, 0)),
        out_shape=jax.ShapeDtypeStruct((N, OUT), jnp.float32),
    )(h0, h1, x, wt, wb)


def kernel(x, edge_index, edge_weight, W):
    src2 = edge_index[1].reshape(NCHUNKS, CH)
    dst2 = edge_index[0].reshape(NCHUNKS, CH)
    wt2 = edge_weight.reshape(NCHUNKS, CH)
    zeros = jnp.zeros((N, D), jnp.float32)
    hp = _sc_aggregate(x, src2, dst2, wt2, zeros)
    return _tc_project(hp[0], hp[1], x, W[:D], W[D:])


# speedup vs baseline: 2.6305x; 1.3063x over previous
import dataclasses
import functools

import jax
import jax.numpy as jnp
from jax import lax
from jax.experimental import pallas as pl
from jax.experimental.pallas import tpu as pltpu
from jax.experimental.pallas import tpu_sc as plsc

N = 10000
E = 320000
D = 128
OUT = 128

NC = 2
NS = 16
NW = NC * NS
CH = 128
NCHUNKS = E // CH
ROWS_PER_SUB = 624
TAIL_ROWS = N - NS * ROWS_PER_SUB
LANES = 16


def _sc_aggregate(x, src2, dst2, wt2, zeros):
    mesh = plsc.VectorSubcoreMesh(core_axis_name="c", subcore_axis_name="s")
    cp = pltpu.CompilerParams()
    if "needs_layout_passes" in pltpu.CompilerParams.__dataclass_fields__:
        cp = dataclasses.replace(cp, needs_layout_passes=False)

    @functools.partial(
        pl.kernel,
        out_type=jax.ShapeDtypeStruct((NC, N, D), jnp.float32),
        mesh=mesh,
        compiler_params=cp,
        scratch_types=[
            pltpu.VMEM((CH,), jnp.int32),
            pltpu.VMEM((CH,), jnp.int32),
            pltpu.VMEM((CH,), jnp.float32),
            pltpu.VMEM((CH,), jnp.int32),
            pltpu.VMEM((CH,), jnp.int32),
            pltpu.VMEM((CH,), jnp.float32),
            pltpu.VMEM((CH, D), jnp.float32),
            pltpu.VMEM((CH, D), jnp.float32),
            pltpu.VMEM_SHARED((N, D), jnp.float32),
            pltpu.SemaphoreType.DMA,
            pltpu.SemaphoreType.DMA,
            pltpu.SemaphoreType.DMA,
            pltpu.SemaphoreType.DMA,
        ],
    )
    def agg(x_hbm, src_hbm, dst_hbm, wt_hbm, z_hbm, hp_hbm,
            src_a, dst_a, wt_a, src_b, dst_b, wt_b, rows_a, rows_b,
            h_sh, gsem_a, gsem_b, ssem_a, ssem_b):
        cid = lax.axis_index("c")
        sid = lax.axis_index("s")
        wid = sid * NC + cid
        row0 = sid * ROWS_PER_SUB

        pltpu.sync_copy(z_hbm.at[pl.ds(row0, ROWS_PER_SUB)],
                        h_sh.at[pl.ds(row0, ROWS_PER_SUB)])

        @pl.when(sid == NS - 1)
        def _zero_tail():
            pltpu.sync_copy(z_hbm.at[pl.ds(NS * ROWS_PER_SUB, TAIL_ROWS)],
                            h_sh.at[pl.ds(NS * ROWS_PER_SUB, TAIL_ROWS)])

        plsc.subcore_barrier()

        def scale_rows(rows_v, wt_v):
            @pl.loop(0, CH, step=4)
            def _edge(e0):
                for jj in range(4):
                    e = e0 + jj
                    w = plsc.load_gather(
                        wt_v, [jnp.full((LANES,), e, jnp.int32)])
                    for dd in range(D // LANES):
                        sl = pl.ds(dd * LANES, LANES)
                        rows_v[e, sl] = rows_v[e, sl] * w

        @pl.loop(wid, NCHUNKS - NW, step=2 * NW)
        def _pair(c):
            pltpu.sync_copy(src_hbm.at[c], src_a)
            pltpu.sync_copy(dst_hbm.at[c], dst_a)
            pltpu.sync_copy(wt_hbm.at[c], wt_a)
            pltpu.async_copy(x_hbm.at[src_a], rows_a, gsem_a)

            @pl.when(c != wid)
            def _drain_prev():
                pltpu.make_async_copy(rows_b, h_sh.at[dst_b], ssem_b).wait()

            cb = c + NW
            pltpu.sync_copy(src_hbm.at[cb], src_b)
            pltpu.sync_copy(dst_hbm.at[cb], dst_b)
            pltpu.sync_copy(wt_hbm.at[cb], wt_b)
            pltpu.make_async_copy(x_hbm.at[src_a], rows_a, gsem_a).wait()
            pltpu.async_copy(x_hbm.at[src_b], rows_b, gsem_b)
            scale_rows(rows_a, wt_a)
            pltpu.async_copy(rows_a, h_sh.at[dst_a], ssem_a, add=True)
            pltpu.make_async_copy(x_hbm.at[src_b], rows_b, gsem_b).wait()
            scale_rows(rows_b, wt_b)
            pltpu.make_async_copy(rows_a, h_sh.at[dst_a], ssem_a).wait()
            pltpu.async_copy(rows_b, h_sh.at[dst_b], ssem_b, add=True)

        pltpu.make_async_copy(rows_b, h_sh.at[dst_b], ssem_b).wait()

        @pl.when(wid < (NCHUNKS - NW * (NCHUNKS // NW)))
        def _tail_chunk():
            c = NW * (NCHUNKS // NW) + wid
            pltpu.sync_copy(src_hbm.at[c], src_a)
            pltpu.sync_copy(dst_hbm.at[c], dst_a)
            pltpu.sync_copy(wt_hbm.at[c], wt_a)
            pltpu.async_copy(x_hbm.at[src_a], rows_a, gsem_a).wait()
            scale_rows(rows_a, wt_a)
            pltpu.sync_copy(rows_a, h_sh.at[dst_a], add=True)

        plsc.subcore_barrier()
        pltpu.sync_copy(h_sh.at[pl.ds(row0, ROWS_PER_SUB)],
                        hp_hbm.at[cid, pl.ds(row0, ROWS_PER_SUB)])

        @pl.when(sid == NS - 1)
        def _flush_tail():
            pltpu.sync_copy(h_sh.at[pl.ds(NS * ROWS_PER_SUB, TAIL_ROWS)],
                            hp_hbm.at[cid, pl.ds(NS * ROWS_PER_SUB, TAIL_ROWS)])

    return agg(x, src2, dst2, wt2, zeros)


def _tc_project(h0, h1, x, wt, wb):
    RB = 1000

    def body(h0_ref, h1_ref, x_ref, wt_ref, wb_ref, o_ref):
        h = h0_ref[...] + h1_ref[...]
        acc = jnp.dot(h, wt_ref[...], preferred_element_type=jnp.float32)
        acc = acc + jnp.dot(x_ref[...], wb_ref[...],
                            preferred_element_type=jnp.float32)
        o_ref[...] = jnp.maximum(acc, 0.0)

    return pl.pallas_call(
        body,
        grid=(N // RB,),
        in_specs=[
            pl.BlockSpec((RB, D), lambda i: (i, 0)),
            pl.BlockSpec((RB, D), lambda i: (i, 0)),
            pl.BlockSpec((RB, D), lambda i: (i, 0)),
            pl.BlockSpec((D, OUT), lambda i: (0, 0)),
            pl.BlockSpec((D, OUT), lambda i: (0, 0)),
        ],
        out_specs=pl.BlockSpec((RB, OUT), lambda i: (i, 0)),
        out_shape=jax.ShapeDtypeStruct((N, OUT), jnp.float32),
    )(h0, h1, x, wt, wb)


def kernel(x, edge_index, edge_weight, W):
    src2 = edge_index[1].reshape(NCHUNKS, CH)
    dst2 = edge_index[0].reshape(NCHUNKS, CH)
    wt2 = edge_weight.reshape(NCHUNKS, CH)
    zeros = jnp.zeros((N, D), jnp.float32)
    hp = _sc_aggregate(x, src2, dst2, wt2, zeros)
    return _tc_project(hp[0], hp[1], x, W[:D], W[D:])


# speedup vs baseline: 3.1603x; 1.2014x over previous
import dataclasses
import functools

import jax
import jax.numpy as jnp
from jax import lax
from jax.experimental import pallas as pl
from jax.experimental.pallas import tpu as pltpu
from jax.experimental.pallas import tpu_sc as plsc

N = 10000
E = 320000
D = 128
OUT = 128

NC = 2
NS = 16
NW = NC * NS
CH = 128
NCHUNKS = E // CH
ROWS_PER_SUB = 624
TAIL_ROWS = N - NS * ROWS_PER_SUB
LANES = 16


def _sc_aggregate(x, src2, dst2, wt2, zeros):
    mesh = plsc.VectorSubcoreMesh(core_axis_name="c", subcore_axis_name="s")
    cp = pltpu.CompilerParams()
    if "needs_layout_passes" in pltpu.CompilerParams.__dataclass_fields__:
        cp = dataclasses.replace(cp, needs_layout_passes=False)

    @functools.partial(
        pl.kernel,
        out_type=jax.ShapeDtypeStruct((NC, N, D), jnp.float32),
        mesh=mesh,
        compiler_params=cp,
        scratch_types=[
            pltpu.VMEM((CH,), jnp.int32),
            pltpu.VMEM((CH,), jnp.int32),
            pltpu.VMEM((CH,), jnp.float32),
            pltpu.VMEM((CH,), jnp.int32),
            pltpu.VMEM((CH,), jnp.int32),
            pltpu.VMEM((CH,), jnp.float32),
            pltpu.VMEM((CH, D), jnp.float32),
            pltpu.VMEM((CH, D), jnp.float32),
            pltpu.VMEM_SHARED((N, D), jnp.float32),
            pltpu.SemaphoreType.DMA,
            pltpu.SemaphoreType.DMA,
            pltpu.SemaphoreType.DMA,
            pltpu.SemaphoreType.DMA,
            pltpu.SemaphoreType.DMA,
            pltpu.SemaphoreType.DMA,
        ],
    )
    def agg(x_hbm, src_hbm, dst_hbm, wt_hbm, z_hbm, hp_hbm,
            src_a, dst_a, wt_a, src_b, dst_b, wt_b, rows_a, rows_b,
            h_sh, gsem_a, gsem_b, ssem_a, ssem_b, isem_a, isem_b):
        cid = lax.axis_index("c")
        sid = lax.axis_index("s")
        wid = sid * NC + cid
        row0 = sid * ROWS_PER_SUB

        pltpu.sync_copy(z_hbm.at[pl.ds(row0, ROWS_PER_SUB)],
                        h_sh.at[pl.ds(row0, ROWS_PER_SUB)])

        @pl.when(sid == NS - 1)
        def _zero_tail():
            pltpu.sync_copy(z_hbm.at[pl.ds(NS * ROWS_PER_SUB, TAIL_ROWS)],
                            h_sh.at[pl.ds(NS * ROWS_PER_SUB, TAIL_ROWS)])

        plsc.subcore_barrier()

        def scale_rows(rows_v, wt_v):
            @pl.loop(0, CH, step=8)
            def _edge(e0):
                for jj in range(8):
                    e = e0 + jj
                    w = plsc.load_gather(
                        wt_v, [jnp.full((LANES,), e, jnp.int32)])
                    for dd in range(D // LANES):
                        sl = pl.ds(dd * LANES, LANES)
                        rows_v[e, sl] = rows_v[e, sl] * w

        @pl.loop(wid, NCHUNKS - NW, step=2 * NW)
        def _pair(c):
            cb = c + NW
            pltpu.sync_copy(src_hbm.at[c], src_a)
            pltpu.async_copy(x_hbm.at[src_a], rows_a, gsem_a)
            pltpu.async_copy(dst_hbm.at[c], dst_a, isem_a)
            pltpu.async_copy(wt_hbm.at[c], wt_a, isem_a)

            @pl.when(c != wid)
            def _drain_prev():
                pltpu.make_async_copy(rows_b, h_sh.at[dst_b], ssem_b).wait()

            pltpu.sync_copy(src_hbm.at[cb], src_b)
            pltpu.async_copy(x_hbm.at[src_b], rows_b, gsem_b)
            pltpu.async_copy(dst_hbm.at[cb], dst_b, isem_b)
            pltpu.async_copy(wt_hbm.at[cb], wt_b, isem_b)

            pltpu.make_async_copy(x_hbm.at[src_a], rows_a, gsem_a).wait()
            pltpu.make_async_copy(dst_hbm.at[c], dst_a, isem_a).wait()
            pltpu.make_async_copy(wt_hbm.at[c], wt_a, isem_a).wait()
            scale_rows(rows_a, wt_a)
            pltpu.async_copy(rows_a, h_sh.at[dst_a], ssem_a, add=True)

            pltpu.make_async_copy(x_hbm.at[src_b], rows_b, gsem_b).wait()
            pltpu.make_async_copy(dst_hbm.at[cb], dst_b, isem_b).wait()
            pltpu.make_async_copy(wt_hbm.at[cb], wt_b, isem_b).wait()
            scale_rows(rows_b, wt_b)
            pltpu.make_async_copy(rows_a, h_sh.at[dst_a], ssem_a).wait()
            pltpu.async_copy(rows_b, h_sh.at[dst_b], ssem_b, add=True)

        pltpu.make_async_copy(rows_b, h_sh.at[dst_b], ssem_b).wait()

        @pl.when(wid < (NCHUNKS - NW * (NCHUNKS // NW)))
        def _tail_chunk():
            c = NW * (NCHUNKS // NW) + wid
            pltpu.sync_copy(src_hbm.at[c], src_a)
            pltpu.sync_copy(dst_hbm.at[c], dst_a)
            pltpu.sync_copy(wt_hbm.at[c], wt_a)
            pltpu.async_copy(x_hbm.at[src_a], rows_a, gsem_a).wait()
            scale_rows(rows_a, wt_a)
            pltpu.sync_copy(rows_a, h_sh.at[dst_a], add=True)

        plsc.subcore_barrier()
        pltpu.sync_copy(h_sh.at[pl.ds(row0, ROWS_PER_SUB)],
                        hp_hbm.at[cid, pl.ds(row0, ROWS_PER_SUB)])

        @pl.when(sid == NS - 1)
        def _flush_tail():
            pltpu.sync_copy(h_sh.at[pl.ds(NS * ROWS_PER_SUB, TAIL_ROWS)],
                            hp_hbm.at[cid, pl.ds(NS * ROWS_PER_SUB, TAIL_ROWS)])

    return agg(x, src2, dst2, wt2, zeros)


def _tc_project(h0, h1, x, wt, wb):
    RB = 1000

    def body(h0_ref, h1_ref, x_ref, wt_ref, wb_ref, o_ref):
        h = h0_ref[...] + h1_ref[...]
        acc = jnp.dot(h, wt_ref[...], preferred_element_type=jnp.float32)
        acc = acc + jnp.dot(x_ref[...], wb_ref[...],
                            preferred_element_type=jnp.float32)
        o_ref[...] = jnp.maximum(acc, 0.0)

    return pl.pallas_call(
        body,
        grid=(N // RB,),
        in_specs=[
            pl.BlockSpec((RB, D), lambda i: (i, 0)),
            pl.BlockSpec((RB, D), lambda i: (i, 0)),
            pl.BlockSpec((RB, D), lambda i: (i, 0)),
            pl.BlockSpec((D, OUT), lambda i: (0, 0)),
            pl.BlockSpec((D, OUT), lambda i: (0, 0)),
        ],
        out_specs=pl.BlockSpec((RB, OUT), lambda i: (i, 0)),
        out_shape=jax.ShapeDtypeStruct((N, OUT), jnp.float32),
    )(h0, h1, x, wt, wb)


def kernel(x, edge_index, edge_weight, W):
    src2 = edge_index[1].reshape(NCHUNKS, CH)
    dst2 = edge_index[0].reshape(NCHUNKS, CH)
    wt2 = edge_weight.reshape(NCHUNKS, CH)
    zeros = jnp.zeros((N, D), jnp.float32)
    hp = _sc_aggregate(x, src2, dst2, wt2, zeros)
    return _tc_project(hp[0], hp[1], x, W[:D], W[D:])
